# baseline (speedup 1.0000x reference)
import functools

import jax
import jax.numpy as jnp
from jax import lax
from jax.experimental import pallas as pl
from jax.experimental.pallas import tpu as pltpu
from jax.experimental.pallas import tpu_sc as plsc

BATCH = 16384
NS = 5
NFACT = 4
NRULES = NS ** NFACT
CT = 5
RPAD = CT * 128
TILE_W = 8 * 128

_info = plsc.get_sparse_core_info()
_NC, _NSUB, _L = _info.num_cores, _info.num_subcores, _info.num_lanes
NW = _NC * _NSUB
ROWS_PER_W = BATCH // NW
RB = 16
NBLK = ROWS_PER_W // RB
MT_W = NFACT * NS * ROWS_PER_W
BUF_W = RB * RPAD


def _sc_call(mt):
    mesh = plsc.VectorSubcoreMesh(core_axis_name="c", subcore_axis_name="s")

    @functools.partial(
        pl.kernel,
        mesh=mesh,
        out_type=jax.ShapeDtypeStruct((BATCH * RPAD,), jnp.float32),
        compiler_params=pltpu.CompilerParams(needs_layout_passes=False),
        scratch_types=[
            pltpu.VMEM((MT_W,), jnp.float32),
            pltpu.VMEM((2 * BUF_W,), jnp.float32),
            pltpu.SemaphoreType.DMA,
            pltpu.SemaphoreType.DMA,
        ],
    )
    def k(mt_hbm, out_hbm, mt_v, buf_v, sem0, sem1):
        wid = lax.axis_index("s") * _NC + lax.axis_index("c")
        pltpu.sync_copy(mt_hbm.at[pl.ds(wid * MT_W, MT_W)], mt_v)
        lane = lax.iota(jnp.int32, _L)
        tile_off = (lane // 8) * (CT * TILE_W) + (lane % 8) * 128

        def _drain(sem):
            pltpu.make_async_copy(buf_v.at[pl.ds(0, BUF_W)],
                                  out_hbm.at[pl.ds(0, BUF_W)], sem).wait()

        def block(t, carry):
            par = jnp.bitwise_and(t, 1)
            base_idx = tile_off + par * BUF_W

            @pl.when(t >= 2)
            def _():
                @pl.when(par == 0)
                def _():
                    _drain(sem0)
                @pl.when(par == 1)
                def _():
                    _drain(sem1)

            vs = [[mt_v[pl.ds((j * NS + i) * ROWS_PER_W + t * RB, RB)]
                   for i in range(NS)] for j in range(NFACT)]
            for i0 in range(NS):
                v0 = vs[0][i0]
                for i1 in range(NS):
                    v01 = v0 * vs[1][i1]
                    for i2 in range(NS):
                        v012 = v01 * vs[2][i2]
                        for i3 in range(NS):
                            r = ((i0 * NS + i1) * NS + i2) * NS + i3
                            roff = (r // 128) * TILE_W + (r % 128)
                            val = v012 * vs[3][i3]
                            plsc.store_scatter(buf_v, [base_idx + roff], val)
            out_off = (wid * ROWS_PER_W + t * RB) * RPAD

            @pl.when(par == 0)
            def _():
                pltpu.async_copy(buf_v.at[pl.ds(0, BUF_W)],
                                 out_hbm.at[pl.ds(out_off, BUF_W)], sem0)

            @pl.when(par == 1)
            def _():
                pltpu.async_copy(buf_v.at[pl.ds(BUF_W, BUF_W)],
                                 out_hbm.at[pl.ds(out_off, BUF_W)], sem1)
            return carry

        lax.fori_loop(0, NBLK, block, 0)
        _drain(sem0)
        _drain(sem1)

    return k(mt)


def kernel(m0, m1, m2, m3):
    mt = jnp.concatenate([m0.T, m1.T, m2.T, m3.T], axis=0)
    mt = mt.reshape(NFACT * NS, NW, ROWS_PER_W).transpose(1, 0, 2)
    flat = _sc_call(mt.reshape(-1))
    y = flat.reshape(BATCH // 8, CT, 8, 128).transpose(0, 2, 1, 3)
    return y.reshape(BATCH, RPAD)[:, :NRULES]


# speedup vs baseline: 1.0947x; 1.0947x over previous
import functools

import jax
import jax.numpy as jnp
from jax import lax
from jax.experimental import pallas as pl
from jax.experimental.pallas import tpu as pltpu
from jax.experimental.pallas import tpu_sc as plsc

BATCH = 16384
NS = 5
NFACT = 4
NRULES = NS ** NFACT

_info = plsc.get_sparse_core_info()
_NC, _NSUB, _L = _info.num_cores, _info.num_subcores, _info.num_lanes
NW = _NC * _NSUB
ROWS_PER_W = BATCH // NW
RB = 16
NBLK = ROWS_PER_W // RB
MT_W = NFACT * NS * ROWS_PER_W
BUF_W = RB * NRULES


def _sc_call(mt):
    mesh = plsc.VectorSubcoreMesh(core_axis_name="c", subcore_axis_name="s")

    @functools.partial(
        pl.kernel,
        mesh=mesh,
        out_type=jax.ShapeDtypeStruct((BATCH, NRULES), jnp.float32),
        compiler_params=pltpu.CompilerParams(needs_layout_passes=False),
        scratch_types=[
            pltpu.VMEM((MT_W,), jnp.float32),
            pltpu.VMEM((2 * RB, NRULES), jnp.float32),
            pltpu.SemaphoreType.DMA,
            pltpu.SemaphoreType.DMA,
        ],
    )
    def k(mt_hbm, out_hbm, mt_v, buf_v, sem0, sem1):
        wid = lax.axis_index("s") * _NC + lax.axis_index("c")
        pltpu.sync_copy(mt_hbm.at[pl.ds(wid * MT_W, MT_W)], mt_v)
        lane = lax.iota(jnp.int32, _L)

        def _drain(sem):
            for _ in range(2):
                pltpu.make_async_copy(buf_v.at[pl.ds(0, 8)],
                                      out_hbm.at[pl.ds(0, 8)], sem).wait()

        def block(t, carry):
            par = jnp.bitwise_and(t, 1)
            row_idx = lane + par * RB

            @pl.when(t >= 2)
            def _():
                @pl.when(par == 0)
                def _():
                    _drain(sem0)
                @pl.when(par == 1)
                def _():
                    _drain(sem1)

            vs = [[mt_v[pl.ds((j * NS + i) * ROWS_PER_W + t * RB, RB)]
                   for i in range(NS)] for j in range(NFACT)]
            for i0 in range(NS):
                v0 = vs[0][i0]
                for i1 in range(NS):
                    v01 = v0 * vs[1][i1]
                    for i2 in range(NS):
                        v012 = v01 * vs[2][i2]
                        for i3 in range(NS):
                            r = ((i0 * NS + i1) * NS + i2) * NS + i3
                            val = v012 * vs[3][i3]
                            rvec = jnp.full((_L,), r, jnp.int32)
                            plsc.store_scatter(buf_v, [row_idx, rvec], val)
            row0 = wid * ROWS_PER_W + t * RB

            @pl.when(par == 0)
            def _():
                pltpu.async_copy(buf_v.at[pl.ds(0, 8)],
                                 out_hbm.at[pl.ds(row0, 8)], sem0)
                pltpu.async_copy(buf_v.at[pl.ds(8, 8)],
                                 out_hbm.at[pl.ds(row0 + 8, 8)], sem0)

            @pl.when(par == 1)
            def _():
                pltpu.async_copy(buf_v.at[pl.ds(RB, 8)],
                                 out_hbm.at[pl.ds(row0, 8)], sem1)
                pltpu.async_copy(buf_v.at[pl.ds(RB + 8, 8)],
                                 out_hbm.at[pl.ds(row0 + 8, 8)], sem1)
            return carry

        lax.fori_loop(0, NBLK, block, 0)
        _drain(sem0)
        _drain(sem1)

    return k(mt)


def kernel(m0, m1, m2, m3):
    mt = jnp.concatenate([m0.T, m1.T, m2.T, m3.T], axis=0)
    mt = mt.reshape(NFACT * NS, NW, ROWS_PER_W).transpose(1, 0, 2)
    return _sc_call(mt.reshape(-1))


# speedup vs baseline: 1.8031x; 1.6471x over previous
import functools

import jax
import jax.numpy as jnp
from jax import lax
from jax.experimental import pallas as pl
from jax.experimental.pallas import tpu as pltpu
from jax.experimental.pallas import tpu_sc as plsc

BATCH = 16384
NS = 5
NFACT = 4
NRULES = NS ** NFACT

_info = plsc.get_sparse_core_info()
_NC, _NSUB, _L = _info.num_cores, _info.num_subcores, _info.num_lanes
NW = _NC * _NSUB
ROWS_PER_W = BATCH // NW
RB = 16
NBLK = ROWS_PER_W // RB
MT_W = NFACT * NS * ROWS_PER_W
BUF_W = RB * NRULES


def _sc_call(mt):
    mesh = plsc.VectorSubcoreMesh(core_axis_name="c", subcore_axis_name="s")

    @functools.partial(
        pl.kernel,
        mesh=mesh,
        out_type=jax.ShapeDtypeStruct((BATCH, NRULES), jnp.float32),
        compiler_params=pltpu.CompilerParams(needs_layout_passes=False,
                                            use_tc_tiling_on_sc=False),
        scratch_types=[
            pltpu.VMEM((MT_W,), jnp.float32),
            pltpu.VMEM((2 * RB, NRULES), jnp.float32),
            pltpu.SemaphoreType.DMA,
            pltpu.SemaphoreType.DMA,
        ],
    )
    def k(mt_hbm, out_hbm, mt_v, buf_v, sem0, sem1):
        wid = lax.axis_index("s") * _NC + lax.axis_index("c")
        pltpu.sync_copy(mt_hbm.at[pl.ds(wid * MT_W, MT_W)], mt_v)
        lane = lax.iota(jnp.int32, _L)

        def _drain(sem):
            for _ in range(2):
                pltpu.make_async_copy(buf_v.at[pl.ds(0, 8)],
                                      out_hbm.at[pl.ds(0, 8)], sem).wait()

        def block(t, carry):
            par = jnp.bitwise_and(t, 1)
            row_idx = lane + par * RB

            @pl.when(t >= 2)
            def _():
                @pl.when(par == 0)
                def _():
                    _drain(sem0)
                @pl.when(par == 1)
                def _():
                    _drain(sem1)

            vs = [[mt_v[pl.ds((j * NS + i) * ROWS_PER_W + t * RB, RB)]
                   for i in range(NS)] for j in range(NFACT)]
            for i0 in range(NS):
                v0 = vs[0][i0]
                for i1 in range(NS):
                    v01 = v0 * vs[1][i1]
                    for i2 in range(NS):
                        v012 = v01 * vs[2][i2]
                        for i3 in range(NS):
                            r = ((i0 * NS + i1) * NS + i2) * NS + i3
                            val = v012 * vs[3][i3]
                            rvec = jnp.full((_L,), r, jnp.int32)
                            plsc.store_scatter(buf_v, [row_idx, rvec], val)
            row0 = wid * ROWS_PER_W + t * RB

            @pl.when(par == 0)
            def _():
                pltpu.async_copy(buf_v.at[pl.ds(0, 8)],
                                 out_hbm.at[pl.ds(row0, 8)], sem0)
                pltpu.async_copy(buf_v.at[pl.ds(8, 8)],
                                 out_hbm.at[pl.ds(row0 + 8, 8)], sem0)

            @pl.when(par == 1)
            def _():
                pltpu.async_copy(buf_v.at[pl.ds(RB, 8)],
                                 out_hbm.at[pl.ds(row0, 8)], sem1)
                pltpu.async_copy(buf_v.at[pl.ds(RB + 8, 8)],
                                 out_hbm.at[pl.ds(row0 + 8, 8)], sem1)
            return carry

        lax.fori_loop(0, NBLK, block, 0)
        _drain(sem0)
        _drain(sem1)

    return k(mt)


def kernel(m0, m1, m2, m3):
    mt = jnp.concatenate([m0.T, m1.T, m2.T, m3.T], axis=0)
    mt = mt.reshape(NFACT * NS, NW, ROWS_PER_W).transpose(1, 0, 2)
    return _sc_call(mt.reshape(-1))
